# baseline (speedup 1.0000x reference)
import jax
import jax.numpy as jnp
from jax.experimental import pallas as pl
from jax.experimental.pallas import tpu as pltpu

_TV = 512


def _temp_body(x_ref, wl_ref, bl_ref, rt_ref):
    tl = jnp.sum(x_ref[...] * wl_ref[...], axis=1, keepdims=True) + bl_ref[...]
    rt = 1.0 / jax.nn.sigmoid(tl)
    rt_ref[...] = jnp.broadcast_to(rt, rt_ref.shape)


def _main_body(xb_ref, rt_ref, w_ref, b_ref, o_ref):
    acc = jnp.dot(xb_ref[...], w_ref[...].astype(jnp.bfloat16),
                  preferred_element_type=jnp.float32)
    o_ref[...] = (acc + b_ref[...]) * rt_ref[...][:, 0:1]


def kernel(inputs, W, b):
    x = inputs
    n, k = x.shape
    v = W.shape[1] - 1

    xb = x.astype(jnp.bfloat16)
    wl = W[:, -1].reshape(1, k)
    bl = b[-1].reshape(1, 1)
    b2 = b[:-1].reshape(1, v)

    rt = pl.pallas_call(
        _temp_body,
        out_shape=jax.ShapeDtypeStruct((n, 128), jnp.float32),
    )(x, wl, bl)

    out = pl.pallas_call(
        _main_body,
        grid=(pl.cdiv(v, _TV),),
        in_specs=[
            pl.BlockSpec((n, k), lambda j: (0, 0)),
            pl.BlockSpec((n, 128), lambda j: (0, 0)),
            pl.BlockSpec((k, _TV), lambda j: (0, j)),
            pl.BlockSpec((1, _TV), lambda j: (0, j)),
        ],
        out_specs=pl.BlockSpec((n, _TV), lambda j: (0, j)),
        out_shape=jax.ShapeDtypeStruct((n, v), jnp.float32),
        compiler_params=pltpu.CompilerParams(
            dimension_semantics=("parallel",)),
    )(xb, rt, W, b2)
    return out
